# baseline (speedup 1.0000x reference)
import functools
import math

import jax
import jax.numpy as jnp
from jax import lax
from jax.experimental import pallas as pl
from jax.experimental.pallas import tpu as pltpu
from jax.experimental.pallas import tpu_sc as plsc

NC = 2
NS = 16
NW = NC * NS
LANES = 16


def _make_sc_call(n_rows, d, n_chunks, chunk):
    mesh = plsc.VectorSubcoreMesh(core_axis_name="c", subcore_axis_name="s")
    rows_per_w = n_rows // NW
    scale = 1.0 / math.sqrt(d)

    @functools.partial(
        pl.kernel,
        mesh=mesh,
        out_type=jax.ShapeDtypeStruct((n_rows, d), jnp.float32),
        scratch_types=[
            pltpu.VMEM((rows_per_w,), jnp.int32),
            pltpu.VMEM((chunk, d), jnp.float32),
            pltpu.VMEM((chunk, d), jnp.float32),
            pltpu.SemaphoreType.DMA,
        ],
    )
    def sc_call(x_hbm, pos_hbm, pe_hbm, out_hbm, idx_v, xbuf, pebuf, sem):
        wid = lax.axis_index("s") * NC + lax.axis_index("c")
        base = wid * rows_per_w
        pltpu.sync_copy(pos_hbm.at[pl.ds(base, rows_per_w)], idx_v)
        for c in range(n_chunks):
            row0 = base + c * chunk
            pltpu.sync_copy(x_hbm.at[pl.ds(row0, chunk)], xbuf)
            pltpu.async_copy(
                pe_hbm.at[idx_v.at[pl.ds(c * chunk, chunk)]], pebuf, sem
            ).wait()

            def body(r, _):
                for j in range(d // LANES):
                    sl = pl.ds(j * LANES, LANES)
                    xbuf[r, sl] = xbuf[r, sl] + pebuf[r, sl] * scale
                return 0

            lax.fori_loop(0, chunk, body, 0)
            pltpu.sync_copy(xbuf, out_hbm.at[pl.ds(row0, chunk)])

    return sc_call


def kernel(x, pos, pe):
    b, s, d = x.shape
    n_rows = b * s
    x2 = x.reshape(n_rows, d)
    pos1 = pos.reshape(n_rows).astype(jnp.int32)
    chunk = 32
    n_chunks = n_rows // NW // chunk
    out = _make_sc_call(n_rows, d, n_chunks, chunk)(x2, pos1, pe)
    return out.reshape(b, s, d)


# speedup vs baseline: 1.3704x; 1.3704x over previous
import functools
import math

import jax
import jax.numpy as jnp
from jax import lax
from jax.experimental import pallas as pl
from jax.experimental.pallas import tpu as pltpu
from jax.experimental.pallas import tpu_sc as plsc

NC = 2
NS = 16
NW = NC * NS
LANES = 16


def _make_sc_call(n_rows, d, n_chunks, chunk):
    mesh = plsc.VectorSubcoreMesh(core_axis_name="c", subcore_axis_name="s")
    rows_per_w = n_rows // NW
    scale = 1.0 / math.sqrt(d)

    nbuf = 2

    @functools.partial(
        pl.kernel,
        mesh=mesh,
        out_type=jax.ShapeDtypeStruct((n_rows, d), jnp.float32),
        scratch_types=[
            pltpu.VMEM((rows_per_w,), jnp.int32),
            pltpu.VMEM((chunk, d), jnp.float32),
            pltpu.VMEM((chunk, d), jnp.float32),
            pltpu.VMEM((chunk, d), jnp.float32),
            pltpu.VMEM((chunk, d), jnp.float32),
            pltpu.SemaphoreType.DMA,
            pltpu.SemaphoreType.DMA,
            pltpu.SemaphoreType.DMA,
            pltpu.SemaphoreType.DMA,
            pltpu.SemaphoreType.DMA,
            pltpu.SemaphoreType.DMA,
        ],
    )
    def sc_call(x_hbm, pos_hbm, pe_hbm, out_hbm, idx_v,
                xbuf0, xbuf1, pebuf0, pebuf1,
                semx0, semx1, sempe0, sempe1, semo0, semo1):
        wid = lax.axis_index("s") * NC + lax.axis_index("c")
        base = wid * rows_per_w
        xbufs, pebufs = (xbuf0, xbuf1), (pebuf0, pebuf1)
        semx, sempe, semo = (semx0, semx1), (sempe0, sempe1), (semo0, semo1)
        pltpu.sync_copy(pos_hbm.at[pl.ds(base, rows_per_w)], idx_v)

        def issue_in(c):
            slot = c % nbuf
            row0 = base + c * chunk
            cpx = pltpu.async_copy(
                x_hbm.at[pl.ds(row0, chunk)], xbufs[slot], semx[slot])
            cpp = pltpu.async_copy(
                pe_hbm.at[idx_v.at[pl.ds(c * chunk, chunk)]],
                pebufs[slot], sempe[slot])
            return cpx, cpp

        pending_in = {0: issue_in(0)}
        pending_out = {}
        for c in range(n_chunks):
            slot = c % nbuf
            nslot = (c + 1) % nbuf
            if c + 1 - nbuf in pending_out:
                pending_out.pop(c + 1 - nbuf).wait()
            if c + 1 < n_chunks:
                pending_in[c + 1] = issue_in(c + 1)
            cpx, cpp = pending_in.pop(c)
            cpx.wait()
            cpp.wait()
            xbuf, pebuf = xbufs[slot], pebufs[slot]

            def body(r, _):
                for j in range(d // LANES):
                    sl = pl.ds(j * LANES, LANES)
                    xbuf[r, sl] = xbuf[r, sl] + pebuf[r, sl] * scale
                return 0

            lax.fori_loop(0, chunk, body, 0)
            pending_out[c] = pltpu.async_copy(
                xbuf, out_hbm.at[pl.ds(base + c * chunk, chunk)], semo[slot])
        for c in sorted(pending_out):
            pending_out.pop(c).wait()

    return sc_call


def kernel(x, pos, pe):
    b, s, d = x.shape
    n_rows = b * s
    x2 = x.reshape(n_rows, d)
    pos1 = pos.reshape(n_rows).astype(jnp.int32)
    chunk = 16
    n_chunks = n_rows // NW // chunk
    out = _make_sc_call(n_rows, d, n_chunks, chunk)(x2, pos1, pe)
    return out.reshape(b, s, d)
